# baseline (speedup 1.0000x reference)
import functools

import jax
import jax.numpy as jnp
from jax import lax
from jax.experimental import pallas as pl
from jax.experimental.pallas import tpu as pltpu
from jax.experimental.pallas import tpu_sc as plsc

T = 2048
D = 1024
F = 4096
E = 8
K = 2

BLK = 256
P = K * T + E * BLK
NB = P // BLK
FB = 2048
NJ = F // FB
NS = NJ * NB


NTILE = 32
CHUNK = (K * T) // NTILE


def _sc_rowtok_body(pos_hbm, tok_hbm, rt_hbm, idx_v, val_v, zbuf, shared):
    c = lax.axis_index("c")
    s = lax.axis_index("s")
    wid = s * 2 + c
    base = wid * CHUNK
    pltpu.sync_copy(pos_hbm.at[pl.ds(base, CHUNK)], idx_v)
    pltpu.sync_copy(tok_hbm.at[pl.ds(base, CHUNK)], val_v)

    @pl.when(s == 0)
    def _zero_shared():
        zv = jnp.zeros((16,), jnp.int32)
        for v in range(P // 16):
            zbuf[pl.ds(v * 16, 16)] = zv
        pltpu.sync_copy(zbuf, shared)

    plsc.subcore_barrier()
    pltpu.sync_copy(val_v, shared.at[idx_v], add=True)
    plsc.subcore_barrier()

    @pl.when(s == 0)
    def _export():
        pltpu.sync_copy(shared, rt_hbm.at[c])


@jax.jit
def _sc_rowtok(pos, tok):
    kfn = functools.partial(
        pl.kernel,
        mesh=plsc.VectorSubcoreMesh(core_axis_name="c", subcore_axis_name="s"),
        out_type=jax.ShapeDtypeStruct((2, P), jnp.int32),
        scratch_types=[
            pltpu.VMEM((CHUNK,), jnp.int32),
            pltpu.VMEM((CHUNK,), jnp.int32),
            pltpu.VMEM((P,), jnp.int32),
            pltpu.VMEM_SHARED((P,), jnp.int32),
        ],
    )
    return kfn(_sc_rowtok_body)(pos, tok)


def _gelu(h):
    c = 0.7978845608028654
    return 0.5 * h * (1.0 + jnp.tanh(c * (h + 0.044715 * h * h * h)))


def _w1_copy(w1_hbm, w1buf, sem, e, j, slot):
    return pltpu.make_async_copy(
        w1_hbm.at[e, :, pl.ds(j * FB, FB)], w1buf.at[slot], sem.at[slot])


def _w2_copy(w2_hbm, w2buf, sem, e, j, slot):
    return pltpu.make_async_copy(
        w2_hbm.at[e, pl.ds(j * FB, FB), :], w2buf.at[slot], sem.at[slot])


def _ffn_body(be_ref, aux_ref, xs_ref, w1_hbm, b1_ref, w2_hbm, b2_ref,
              out_ref, w1buf, w2buf, sem1, sem2):
    j = pl.program_id(0)
    i = pl.program_id(1)
    s = j * NB + i
    run_start = aux_ref[0, s]
    nxt_e = aux_ref[1, s]
    nxt_j = aux_ref[2, s]
    have_nxt = aux_ref[3, s]
    slot = aux_ref[4, s]
    cur_e = be_ref[i]

    @pl.when(s == 0)
    def _prime():
        _w1_copy(w1_hbm, w1buf, sem1, cur_e, j, 0).start()
        _w2_copy(w2_hbm, w2buf, sem2, cur_e, j, 0).start()

    @pl.when(run_start == 1)
    def _swap():
        _w1_copy(w1_hbm, w1buf, sem1, cur_e, j, slot).wait()
        _w2_copy(w2_hbm, w2buf, sem2, cur_e, j, slot).wait()

        @pl.when(have_nxt == 1)
        def _prefetch_next():
            _w1_copy(w1_hbm, w1buf, sem1, nxt_e, nxt_j, 1 - slot).start()
            _w2_copy(w2_hbm, w2buf, sem2, nxt_e, nxt_j, 1 - slot).start()

    w1 = w1buf[slot].astype(jnp.bfloat16)
    w2 = w2buf[slot].astype(jnp.bfloat16)
    h = jnp.dot(xs_ref[...], w1, preferred_element_type=jnp.float32)
    h = _gelu(h + b1_ref[0, 0]).astype(jnp.bfloat16)
    y = jnp.dot(h, w2, preferred_element_type=jnp.float32)
    out_ref[0] = jnp.where(j == 0, y + b2_ref[0, 0], y).astype(jnp.bfloat16)


@jax.jit
def _ffn(xs, block_e, aux, W1, b1, W2, b2):
    grid_spec = pltpu.PrefetchScalarGridSpec(
        num_scalar_prefetch=2,
        grid=(NJ, NB),
        in_specs=[
            pl.BlockSpec((BLK, D), lambda j, i, be, aux: (i, 0)),
            pl.BlockSpec(memory_space=pl.ANY),
            pl.BlockSpec((1, 1, FB), lambda j, i, be, aux: (be[i], 0, j)),
            pl.BlockSpec(memory_space=pl.ANY),
            pl.BlockSpec((1, 1, D), lambda j, i, be, aux: (be[i], 0, 0)),
        ],
        out_specs=pl.BlockSpec((1, BLK, D), lambda j, i, be, aux: (j, i, 0)),
        scratch_shapes=[
            pltpu.VMEM((2, D, FB), jnp.float32),
            pltpu.VMEM((2, FB, D), jnp.float32),
            pltpu.SemaphoreType.DMA((2,)),
            pltpu.SemaphoreType.DMA((2,)),
        ],
    )
    return pl.pallas_call(
        _ffn_body,
        grid_spec=grid_spec,
        out_shape=jax.ShapeDtypeStruct((NJ, P, D), jnp.bfloat16),
        compiler_params=pltpu.CompilerParams(
            dimension_semantics=("arbitrary", "arbitrary"),
        ),
    )(block_e, aux, xs, W1, b1.reshape(E, 1, F), W2, b2.reshape(E, 1, D))


def kernel(x, gate_W, W1, b1, W2, b2):
    logits = x @ gate_W
    eidx = jnp.arange(E, dtype=jnp.int32)
    i1 = jnp.argmax(logits, axis=-1).astype(jnp.int32)
    m1 = jnp.max(logits, axis=-1)
    masked = jnp.where(eidx[None, :] == i1[:, None], -jnp.inf, logits)
    i2 = jnp.argmax(masked, axis=-1).astype(jnp.int32)
    m2 = jnp.max(masked, axis=-1)
    e2 = jnp.exp(m2 - m1)
    top_w = jnp.stack([1.0 / (1.0 + e2), e2 / (1.0 + e2)], axis=1)

    ef = jnp.stack([i1, i2], axis=1).reshape(-1)
    onehot = (ef[:, None] == eidx[None, :]).astype(jnp.int32)
    csum = jnp.cumsum(onehot, axis=0)
    counts = csum[-1]
    rank = jnp.take_along_axis(csum, ef[:, None], axis=1)[:, 0] - 1
    blocks_e = (counts + BLK - 1) // BLK
    first_block = jnp.concatenate(
        [jnp.zeros((1,), jnp.int32), jnp.cumsum(blocks_e)[:-1]])
    pad_start = first_block * BLK

    pos = pad_start[ef] + rank
    rt = _sc_rowtok(pos, jnp.arange(K * T, dtype=jnp.int32) // K)
    rowtok = rt[0] + rt[1]

    block_e = (jnp.sum(jnp.arange(NB, dtype=jnp.int32)[:, None]
                       >= first_block[None, :], axis=1) - 1).astype(jnp.int32)

    be_flat = jnp.tile(block_e, NJ)
    sj = jnp.arange(NS, dtype=jnp.int32) // NB
    si = jnp.arange(NS, dtype=jnp.int32) % NB
    rs = ((si == 0) | (be_flat != jnp.roll(be_flat, 1))).astype(jnp.int32)
    g = jnp.cumsum(rs) - 1
    n_runs = g[-1] + 1
    run_e = jnp.zeros((NS + 1,), jnp.int32).at[g].set(be_flat)
    run_j = jnp.zeros((NS + 1,), jnp.int32).at[g].set(sj)
    nxt_e = run_e[jnp.minimum(g + 1, NS)]
    nxt_j = run_j[jnp.minimum(g + 1, NS)]
    have_nxt = (g + 1 < n_runs).astype(jnp.int32)
    slot = g % 2
    aux = jnp.stack([rs, nxt_e, nxt_j, have_nxt, slot], axis=0)

    xs = jnp.take(x.astype(jnp.bfloat16), rowtok, axis=0)
    yp = _ffn(xs, block_e, aux, W1, b1, W2, b2)

    pos2 = pos.reshape(T, K)
    idx2 = jnp.concatenate([pos2[:, 0], pos2[:, 1]])
    yt = (jnp.take(yp[0], idx2, axis=0).astype(jnp.float32)
          + jnp.take(yp[1], idx2, axis=0).astype(jnp.float32))
    return yt[:T] * top_w[:, 0:1] + yt[T:] * top_w[:, 1:2]


# speedup vs baseline: 1.1529x; 1.1529x over previous
import functools

import jax
import jax.numpy as jnp
from jax import lax
from jax.experimental import pallas as pl
from jax.experimental.pallas import tpu as pltpu
from jax.experimental.pallas import tpu_sc as plsc

T = 2048
D = 1024
F = 4096
E = 8
K = 2

BLK = 256
P = K * T + E * BLK
NB = P // BLK
FB = 2048
NJ = F // FB
NS = NJ * NB


NTILE = 32
CHUNK = (K * T) // NTILE


def _sc_rowtok_body(pos_hbm, tok_hbm, rt_hbm, idx_v, val_v, zbuf, shared):
    c = lax.axis_index("c")
    s = lax.axis_index("s")
    wid = s * 2 + c
    base = wid * CHUNK
    pltpu.sync_copy(pos_hbm.at[pl.ds(base, CHUNK)], idx_v)
    pltpu.sync_copy(tok_hbm.at[pl.ds(base, CHUNK)], val_v)

    @pl.when(s == 0)
    def _zero_shared():
        zv = jnp.zeros((16,), jnp.int32)
        for v in range(P // 16):
            zbuf[pl.ds(v * 16, 16)] = zv
        pltpu.sync_copy(zbuf, shared)

    plsc.subcore_barrier()
    pltpu.sync_copy(val_v, shared.at[idx_v], add=True)
    plsc.subcore_barrier()

    @pl.when(s == 0)
    def _export():
        pltpu.sync_copy(shared, rt_hbm.at[c])


@jax.jit
def _sc_rowtok(pos, tok):
    kfn = functools.partial(
        pl.kernel,
        mesh=plsc.VectorSubcoreMesh(core_axis_name="c", subcore_axis_name="s"),
        out_type=jax.ShapeDtypeStruct((2, P), jnp.int32),
        scratch_types=[
            pltpu.VMEM((CHUNK,), jnp.int32),
            pltpu.VMEM((CHUNK,), jnp.int32),
            pltpu.VMEM((P,), jnp.int32),
            pltpu.VMEM_SHARED((P,), jnp.int32),
        ],
    )
    return kfn(_sc_rowtok_body)(pos, tok)


def _gelu(h):
    c = 0.7978845608028654
    return 0.5 * h * (1.0 + jnp.tanh(c * (h + 0.044715 * h * h * h)))


def _w1_copy(w1_hbm, w1buf, sem, e, j, slot):
    return pltpu.make_async_copy(
        w1_hbm.at[e, :, pl.ds(j * FB, FB)], w1buf.at[slot], sem.at[slot])


def _w2_copy(w2_hbm, w2buf, sem, e, j, slot):
    return pltpu.make_async_copy(
        w2_hbm.at[e, pl.ds(j * FB, FB), :], w2buf.at[slot], sem.at[slot])


def _ffn_body(be_ref, aux_ref, xs_ref, w1_hbm, b1_ref, w2_hbm, b2_ref,
              out_ref, w1buf, w2buf, sem1, sem2):
    j = pl.program_id(0)
    i = pl.program_id(1)
    s = j * NB + i
    run_start = aux_ref[0, s]
    nxt_e = aux_ref[1, s]
    nxt_j = aux_ref[2, s]
    have_nxt = aux_ref[3, s]
    slot = aux_ref[4, s]
    cur_e = be_ref[i]

    @pl.when(s == 0)
    def _prime():
        _w1_copy(w1_hbm, w1buf, sem1, cur_e, j, 0).start()
        _w2_copy(w2_hbm, w2buf, sem2, cur_e, j, 0).start()

    @pl.when(run_start == 1)
    def _swap():
        _w1_copy(w1_hbm, w1buf, sem1, cur_e, j, slot).wait()
        _w2_copy(w2_hbm, w2buf, sem2, cur_e, j, slot).wait()

        @pl.when(have_nxt == 1)
        def _prefetch_next():
            _w1_copy(w1_hbm, w1buf, sem1, nxt_e, nxt_j, 1 - slot).start()
            _w2_copy(w2_hbm, w2buf, sem2, nxt_e, nxt_j, 1 - slot).start()

    @pl.when(i < aux_ref[5, s])
    def _compute():
        w1 = w1buf[slot].astype(jnp.bfloat16)
        w2 = w2buf[slot].astype(jnp.bfloat16)
        h = jnp.dot(xs_ref[...], w1, preferred_element_type=jnp.float32)
        h = _gelu(h + b1_ref[0, 0]).astype(jnp.bfloat16)
        y = jnp.dot(h, w2, preferred_element_type=jnp.float32)
        out_ref[0] = jnp.where(j == 0, y + b2_ref[0, 0],
                               y).astype(jnp.bfloat16)


@jax.jit
def _ffn(xs, block_e, aux, W1, b1, W2, b2):
    grid_spec = pltpu.PrefetchScalarGridSpec(
        num_scalar_prefetch=2,
        grid=(NJ, NB),
        in_specs=[
            pl.BlockSpec((BLK, D), lambda j, i, be, aux: (i, 0)),
            pl.BlockSpec(memory_space=pl.ANY),
            pl.BlockSpec((1, 1, FB), lambda j, i, be, aux: (be[i], 0, j)),
            pl.BlockSpec(memory_space=pl.ANY),
            pl.BlockSpec((1, 1, D), lambda j, i, be, aux: (be[i], 0, 0)),
        ],
        out_specs=pl.BlockSpec((1, BLK, D), lambda j, i, be, aux: (j, i, 0)),
        scratch_shapes=[
            pltpu.VMEM((2, D, FB), jnp.float32),
            pltpu.VMEM((2, FB, D), jnp.float32),
            pltpu.SemaphoreType.DMA((2,)),
            pltpu.SemaphoreType.DMA((2,)),
        ],
    )
    return pl.pallas_call(
        _ffn_body,
        grid_spec=grid_spec,
        out_shape=jax.ShapeDtypeStruct((NJ, P, D), jnp.bfloat16),
        compiler_params=pltpu.CompilerParams(
            dimension_semantics=("arbitrary", "arbitrary"),
        ),
    )(block_e, aux, xs, W1, b1.reshape(E, 1, F), W2, b2.reshape(E, 1, D))


def kernel(x, gate_W, W1, b1, W2, b2):
    logits = x @ gate_W
    eidx = jnp.arange(E, dtype=jnp.int32)
    i1 = jnp.argmax(logits, axis=-1).astype(jnp.int32)
    m1 = jnp.max(logits, axis=-1)
    masked = jnp.where(eidx[None, :] == i1[:, None], -jnp.inf, logits)
    i2 = jnp.argmax(masked, axis=-1).astype(jnp.int32)
    m2 = jnp.max(masked, axis=-1)
    e2 = jnp.exp(m2 - m1)
    top_w = jnp.stack([1.0 / (1.0 + e2), e2 / (1.0 + e2)], axis=1)

    ef = jnp.stack([i1, i2], axis=1).reshape(-1)
    onehot = (ef[:, None] == eidx[None, :]).astype(jnp.int32)
    csum = jnp.cumsum(onehot, axis=0)
    counts = csum[-1]
    rank = jnp.take_along_axis(csum, ef[:, None], axis=1)[:, 0] - 1
    blocks_e = (counts + BLK - 1) // BLK
    first_block = jnp.concatenate(
        [jnp.zeros((1,), jnp.int32), jnp.cumsum(blocks_e)[:-1]])
    pad_start = first_block * BLK

    pos = pad_start[ef] + rank
    rt = _sc_rowtok(pos, jnp.arange(K * T, dtype=jnp.int32) // K)
    rowtok = rt[0] + rt[1]

    block_e = (jnp.sum(jnp.arange(NB, dtype=jnp.int32)[:, None]
                       >= first_block[None, :], axis=1) - 1).astype(jnp.int32)

    be_flat = jnp.tile(block_e, NJ)
    sj = jnp.arange(NS, dtype=jnp.int32) // NB
    si = jnp.arange(NS, dtype=jnp.int32) % NB
    rs = ((si == 0) | (be_flat != jnp.roll(be_flat, 1))).astype(jnp.int32)
    g = jnp.cumsum(rs) - 1
    n_runs = g[-1] + 1
    run_e = jnp.zeros((NS + 1,), jnp.int32).at[g].set(be_flat)
    run_j = jnp.zeros((NS + 1,), jnp.int32).at[g].set(sj)
    nxt_e = run_e[jnp.minimum(g + 1, NS)]
    nxt_j = run_j[jnp.minimum(g + 1, NS)]
    have_nxt = (g + 1 < n_runs).astype(jnp.int32)
    slot = g % 2
    n_used = jnp.broadcast_to(first_block[E - 1] + blocks_e[E - 1], (NS,))
    aux = jnp.stack([rs, nxt_e, nxt_j, have_nxt, slot, n_used], axis=0)

    xs = jnp.take(x.astype(jnp.bfloat16), rowtok, axis=0)
    yp = _ffn(xs, block_e, aux, W1, b1, W2, b2)
    ys = (yp[0].astype(jnp.float32) + yp[1].astype(jnp.float32)
          ).astype(jnp.bfloat16)

    pos2 = pos.reshape(T, K)
    yt = jnp.take(ys, jnp.concatenate([pos2[:, 0], pos2[:, 1]]), axis=0)
    yt = yt.astype(jnp.float32)
    return yt[:T] * top_w[:, 0:1] + yt[T:] * top_w[:, 1:2]
